# Initial kernel scaffold; baseline (speedup 1.0000x reference)
import jax
import jax.numpy as jnp
from jax.experimental import pallas as pl


def kernel(x, edge_index, le_adj, W1, b1, W2, b2):
    raise NotImplementedError("write your pallas kernel here")


# speedup vs baseline: 3.7770x; 3.7770x over previous
import jax
import jax.numpy as jnp
from jax import lax
from jax.experimental import pallas as pl
from jax.experimental.pallas import tpu as pltpu
from jax.experimental.pallas import tpu_sc as plsc

N = 10000
P = 160000
E = 640000
D = 128
DC = 40
DCP = 48
NP = 10240
ET = E + P
ET_PAD = 819200
EPT = ET_PAD // 32
PH_PAD = 163840
NC, NS = 2, 16

R1 = 10240
NP1 = 8
S1 = R1 // NS
R2 = 16384
NP2 = 5
S2 = R2 // NS
CE = 1024
EPS = ET_PAD // NS
NCH = EPS // CE
NW = CE // 128

_mesh = plsc.VectorSubcoreMesh(core_axis_name="c", subcore_axis_name="s")
_params = pltpu.CompilerParams(needs_layout_passes=False)


def _sload(ref, i):
  return ref[pl.ds(i, 16)][0]


def _zero_rows(ref, nrows, width):
  def body(k, _):
    for f in range(width // 16):
      ref[k, pl.ds(f * 16, 16)] = jnp.zeros((16,), jnp.float32)
    return 0
  lax.fori_loop(0, nrows, body, 0)


def _prep_body(dst_all, src0p, srcle_all, src0t, hist2, cnt2, comp,
               histP_s, histN_s, zbuf, dbuf, obuf, sbuf, cobuf):
  c = lax.axis_index("c")
  s = lax.axis_index("s")
  wid = s * NC + c

  def zb(i, _):
    zbuf[pl.ds(i * 16, 16)] = jnp.zeros((16,), jnp.float32)
    return 0
  lax.fori_loop(0, 125, zb, 0)
  for q in range(5):
    pltpu.sync_copy(zbuf, histP_s.at[pl.ds(s * 10000 + q * 2000, 2000)])
  pltpu.sync_copy(zbuf.at[pl.ds(0, NP // NS)], histN_s.at[pl.ds(s * (NP // NS), NP // NS)])
  plsc.subcore_barrier()

  base = wid * EPT
  def chunk(i, _):
    off = base + i * 128
    pltpu.sync_copy(dst_all.at[pl.ds(off, 128)], dbuf)
    for j in range(8):
      gi = off + j * 16 + lax.iota(jnp.int32, 16)
      obuf[pl.ds(j * 16, 16)] = jnp.where(gi < ET, 1.0, 0.0).astype(jnp.float32)
    pltpu.sync_copy(obuf, histP_s.at[dbuf], add=True)
    pltpu.sync_copy(srcle_all.at[pl.ds(off, 128)], sbuf)
    pltpu.sync_copy(src0t.at[sbuf], cobuf)
    pltpu.sync_copy(cobuf, comp.at[pl.ds(off, 128)])
    return 0
  lax.fori_loop(0, EPT // 128, chunk, 0)

  base2 = wid * (PH_PAD // 32)
  def chunk2(i, _):
    off = base2 + i * 128
    pltpu.sync_copy(src0p.at[pl.ds(off, 128)], dbuf)
    for j in range(8):
      gi = off + j * 16 + lax.iota(jnp.int32, 16)
      obuf[pl.ds(j * 16, 16)] = jnp.where(gi < P, 1.0, 0.0).astype(jnp.float32)
    pltpu.sync_copy(obuf, histN_s.at[dbuf], add=True)
    return 0
  lax.fori_loop(0, PH_PAD // 32 // 128, chunk2, 0)
  plsc.subcore_barrier()

  for q in range(5):
    pltpu.sync_copy(histP_s.at[pl.ds(s * 10000 + q * 2000, 2000)], zbuf)
    pltpu.sync_copy(zbuf, hist2.at[pl.ds(c * P + s * 10000 + q * 2000, 2000)])
  pltpu.sync_copy(histN_s.at[pl.ds(s * (NP // NS), NP // NS)], zbuf.at[pl.ds(0, NP // NS)])
  pltpu.sync_copy(zbuf.at[pl.ds(0, NP // NS)],
                  cnt2.at[pl.ds(c * NP + s * (NP // NS), NP // NS)])


_prep = pl.kernel(
    _prep_body,
    out_type=[jax.ShapeDtypeStruct((2 * P,), jnp.float32),
              jax.ShapeDtypeStruct((2 * NP,), jnp.float32),
              jax.ShapeDtypeStruct((ET_PAD,), jnp.int32)],
    mesh=_mesh,
    scratch_types=[pltpu.VMEM_SHARED((P,), jnp.float32),
                   pltpu.VMEM_SHARED((NP,), jnp.float32),
                   pltpu.VMEM((2000,), jnp.float32),
                   pltpu.VMEM((128,), jnp.int32),
                   pltpu.VMEM((128,), jnp.float32),
                   pltpu.VMEM((128,), jnp.int32),
                   pltpu.VMEM((128,), jnp.int32)],
    compiler_params=_params)


def _msg_passes(*, c, s, npass, rr, table, dst_all, comp_or_src, wsrc, acc,
                dbuf, cbuf, wbuf, cidx, cw, cdst, rows, sem, width, epilogue):
  base = s * EPS

  def do_pass(r, _):
    lo = (c * npass + r) * rr
    _zero_rows(rows, 128, width)
    for q in range(rr // NS // 128):
      pltpu.sync_copy(rows.at[pl.ds(0, 128)],
                      acc.at[pl.ds(s * (rr // NS) + q * 128, 128)])
    plsc.subcore_barrier()

    def chunk(i, _):
      choff = base + i * CE
      d1 = pltpu.make_async_copy(dst_all.at[pl.ds(choff, CE)], dbuf, sem)
      d2 = pltpu.make_async_copy(comp_or_src.at[pl.ds(choff, CE)], cbuf, sem)
      d3 = pltpu.make_async_copy(wsrc.at[pl.ds(choff, CE)], wbuf.at[pl.ds(0, CE)], sem)
      d1.start(); d2.start(); d3.start()
      d1.wait(); d2.wait(); d3.wait()
      def zcw(j, _):
        cw[pl.ds(j * 16, 16)] = jnp.zeros((16,), jnp.float32)
        return 0
      lax.fori_loop(0, CE // 16, zcw, 0)
      cnt = jnp.int32(0)
      for j in range(CE // 16):
        off = j * 16
        d16 = dbuf[pl.ds(off, 16)]
        gi = choff + off + lax.iota(jnp.int32, 16)
        inb = (d16 >= lo) & (d16 < lo + rr) & (gi < ET)
        ii = inb.astype(jnp.int32)
        pos = cnt + plsc.cumsum(ii) - 1
        plsc.store_scatter(cidx, [pos], cbuf[pl.ds(off, 16)], mask=inb)
        plsc.store_scatter(cw, [pos], wbuf[pl.ds(off, 16)], mask=inb)
        plsc.store_scatter(cdst, [pos], d16 - lo, mask=inb)
        cnt = cnt + jnp.sum(ii)
      for w in range(NW):
        wb = w * 128
        @pl.when(wb < cnt)
        def _wave():
          for b in range(8):
            gb = wb + b * 16
            @pl.when(gb < cnt)
            def _g():
              i16 = cidx[pl.ds(gb, 16)]
              pltpu.async_copy(table.at[i16], rows.at[pl.ds(b * 16, 16)], sem)
          for b in range(8):
            gb = wb + b * 16
            @pl.when(gb < cnt)
            def _gw():
              i16 = cidx[pl.ds(gb, 16)]
              pltpu.make_async_copy(table.at[i16], rows.at[pl.ds(b * 16, 16)], sem).wait()
          nrow = jnp.minimum(cnt - wb, 128)
          ng = (nrow + 15) // 16 * 16
          def sc_row(k, _):
            wk = _sload(cw, wb + k)
            for f in range(width // 16):
              rows[k, pl.ds(f * 16, 16)] = rows[k, pl.ds(f * 16, 16)] * wk
            return 0
          lax.fori_loop(0, ng, sc_row, 0)
          for b in range(8):
            gb = wb + b * 16
            @pl.when(gb < cnt)
            def _s():
              d16 = cdst[pl.ds(gb, 16)]
              pltpu.async_copy(rows.at[pl.ds(b * 16, 16)], acc.at[d16], sem, add=True)
          for b in range(8):
            gb = wb + b * 16
            @pl.when(gb < cnt)
            def _sw():
              d16 = cdst[pl.ds(gb, 16)]
              pltpu.make_async_copy(rows.at[pl.ds(b * 16, 16)], acc.at[d16], sem).wait()
      return 0
    lax.fori_loop(0, NCH, chunk, 0)
    plsc.subcore_barrier()
    epilogue(lo)
    plsc.subcore_barrier()
    return 0
  lax.fori_loop(0, npass, do_pass, 0)


def _l1_body(dst_all, comp, srcle_all, dinv, xw1, b1, h1, wout,
             acc, dbuf, cbuf, wbuf, cidx, cw, cdst, rows, dinvv, b1v, tbuf, sem):
  c = lax.axis_index("c")
  s = lax.axis_index("s")
  base = s * EPS

  pltpu.sync_copy(b1, b1v)
  def p0(i, _):
    off = base + i * CE
    pltpu.sync_copy(srcle_all.at[pl.ds(off, CE)], cbuf)
    for j in range(CE // 128):
      pltpu.sync_copy(dinv.at[cbuf.at[pl.ds(j * 128, 128)]],
                      wbuf.at[pl.ds(j * 128, 128)])
    pltpu.sync_copy(wbuf.at[pl.ds(0, CE)], wout.at[pl.ds(off, CE)])
    return 0
  lax.fori_loop(0, NCH, p0, 0)

  def zc(j, _):
    cidx[pl.ds(j * 16, 16)] = jnp.zeros((16,), jnp.int32)
    cdst[pl.ds(j * 16, 16)] = jnp.zeros((16,), jnp.int32)
    return 0
  lax.fori_loop(0, CE // 16, zc, 0)

  def epilogue(lo):
    @pl.when(lo + s * S1 < P)
    def _out():
      pltpu.sync_copy(dinv.at[pl.ds(lo + s * S1, S1)], dinvv.at[pl.ds(0, S1)])
      for q in range(S1 // 128):
        g0 = lo + s * S1 + q * 128
        pltpu.sync_copy(acc.at[pl.ds(s * S1 + q * 128, 128)], rows.at[pl.ds(0, 128)])
        def sk(k, _):
          dk = _sload(dinvv, q * 128 + k)
          for f in range(D // 16):
            v = rows[k, pl.ds(f * 16, 16)] * dk + b1v[pl.ds(f * 16, 16)]
            rows[k, pl.ds(f * 16, 16)] = jnp.maximum(v, 0.0)
          return 0
        lax.fori_loop(0, 128, sk, 0)
        pltpu.sync_copy(rows.at[pl.ds(0, 128)], h1.at[pl.ds(g0, 128)])

  _msg_passes(c=c, s=s, npass=NP1, rr=R1, table=xw1, dst_all=dst_all,
              comp_or_src=comp, wsrc=wout, acc=acc, dbuf=dbuf, cbuf=cbuf,
              wbuf=wbuf, cidx=cidx, cw=cw, cdst=cdst, rows=rows, sem=sem,
              width=D, epilogue=epilogue)


_l1 = pl.kernel(
    _l1_body,
    out_type=[jax.ShapeDtypeStruct((P, D), jnp.float32),
              jax.ShapeDtypeStruct((ET_PAD,), jnp.float32)],
    mesh=_mesh,
    scratch_types=[pltpu.VMEM_SHARED((R1, D), jnp.float32),
                   pltpu.VMEM((CE,), jnp.int32),
                   pltpu.VMEM((CE,), jnp.int32),
                   pltpu.VMEM((CE + 16,), jnp.float32),
                   pltpu.VMEM((CE,), jnp.int32),
                   pltpu.VMEM((CE + 16,), jnp.float32),
                   pltpu.VMEM((CE,), jnp.int32),
                   pltpu.VMEM((128, D), jnp.float32),
                   pltpu.VMEM((S1 + 16,), jnp.float32),
                   pltpu.VMEM((D,), jnp.float32),
                   pltpu.VMEM((128,), jnp.int32),
                   pltpu.SemaphoreType.DMA],
    compiler_params=_params)


def _l2_body(dst_all, srcle_all, src0t, dinv, z, b2p, wsrc, out2,
             acc, outacc, dbuf, cbuf, wbuf, cidx, cw, cdst, rows,
             dinvv, b2v, src0v, h2buf, sem):
  c = lax.axis_index("c")
  s = lax.axis_index("s")

  pltpu.sync_copy(b2p, b2v)
  _zero_rows(h2buf, 128, DCP)
  for q in range(NP // NS // 128):
    pltpu.sync_copy(h2buf.at[pl.ds(0, 128)],
                    outacc.at[pl.ds(s * (NP // NS) + q * 128, 128)])

  def zc(j, _):
    cidx[pl.ds(j * 16, 16)] = jnp.zeros((16,), jnp.int32)
    cdst[pl.ds(j * 16, 16)] = jnp.zeros((16,), jnp.int32)
    return 0
  lax.fori_loop(0, CE // 16, zc, 0)

  def epilogue(lo):
    for q in range(S2 // 256):
      l0 = s * S2 + q * 256
      g0 = lo + l0
      @pl.when(g0 < P)
      def _blk():
        pltpu.sync_copy(acc.at[pl.ds(l0, 256)], h2buf)
        pltpu.sync_copy(dinv.at[pl.ds(g0, 256)], dinvv.at[pl.ds(0, 256)])
        pltpu.sync_copy(src0t.at[pl.ds(g0, 256)], src0v)
        def sk(k, _):
          dk = _sload(dinvv, k)
          for f in range(DCP // 16):
            h2buf[k, pl.ds(f * 16, 16)] = (h2buf[k, pl.ds(f * 16, 16)] * dk
                                           + b2v[pl.ds(f * 16, 16)])
          return 0
        lax.fori_loop(0, 256, sk, 0)
        for g in range(16):
          n16 = src0v[pl.ds(g * 16, 16)]
          pltpu.sync_copy(h2buf.at[pl.ds(g * 16, 16)], outacc.at[n16], add=True)

  _msg_passes(c=c, s=s, npass=NP2, rr=R2, table=z, dst_all=dst_all,
              comp_or_src=srcle_all, wsrc=wsrc, acc=acc, dbuf=dbuf, cbuf=cbuf,
              wbuf=wbuf, cidx=cidx, cw=cw, cdst=cdst, rows=rows, sem=sem,
              width=DCP, epilogue=epilogue)

  for q in range(5):
    l0 = s * (NP // NS) + q * 128
    pltpu.sync_copy(outacc.at[pl.ds(l0, 128)], h2buf.at[pl.ds(0, 128)])
    pltpu.sync_copy(h2buf.at[pl.ds(0, 128)], out2.at[pl.ds(c * NP + l0, 128)])


_l2 = pl.kernel(
    _l2_body,
    out_type=jax.ShapeDtypeStruct((2 * NP, DCP), jnp.float32),
    mesh=_mesh,
    scratch_types=[pltpu.VMEM_SHARED((R2, DCP), jnp.float32),
                   pltpu.VMEM_SHARED((NP, DCP), jnp.float32),
                   pltpu.VMEM((CE,), jnp.int32),
                   pltpu.VMEM((CE,), jnp.int32),
                   pltpu.VMEM((CE + 16,), jnp.float32),
                   pltpu.VMEM((CE,), jnp.int32),
                   pltpu.VMEM((CE + 16,), jnp.float32),
                   pltpu.VMEM((CE,), jnp.int32),
                   pltpu.VMEM((128, DCP), jnp.float32),
                   pltpu.VMEM((256 + 16,), jnp.float32),
                   pltpu.VMEM((DCP,), jnp.float32),
                   pltpu.VMEM((256,), jnp.int32),
                   pltpu.VMEM((256, DCP), jnp.float32),
                   pltpu.SemaphoreType.DMA],
    compiler_params=pltpu.CompilerParams(needs_layout_passes=False,
                                         use_tc_tiling_on_sc=False))


def _dinv_body(h_ref, o_ref):
  o_ref[...] = lax.rsqrt(jnp.maximum(h_ref[0] + h_ref[1], 1e-12))


def _mm_body(x_ref, w_ref, o_ref):
  o_ref[...] = jnp.dot(x_ref[...], w_ref[...], preferred_element_type=jnp.float32)


def _fin_body(a_ref, b_ref, ca_ref, cb_ref, o_ref):
  cnt = ca_ref[...] + cb_ref[...]
  s = a_ref[...] + b_ref[...]
  o_ref[...] = jnp.where(cnt > 0, s / jnp.maximum(cnt, 1.0), 0.0)


def kernel(x, edge_index, le_adj, W1, b1, W2, b2):
  src0 = edge_index[0].astype(jnp.int32)
  src_le = le_adj[0].astype(jnp.int32)
  dst_le = le_adj[1].astype(jnp.int32)
  iota_p = jnp.arange(P, dtype=jnp.int32)
  padE = jnp.zeros((ET_PAD - ET,), jnp.int32)
  dst_all = jnp.concatenate([dst_le, iota_p, padE])
  srcle_all = jnp.concatenate([src_le, iota_p, padE])
  src0p = jnp.concatenate([src0, jnp.zeros((PH_PAD - P,), jnp.int32)])
  W2p = jnp.pad(W2, ((0, 0), (0, DCP - DC)))
  b2p = jnp.pad(b2, (0, DCP - DC))

  hist2, cnt2, comp = _prep(dst_all, src0p, srcle_all, src0)

  dinv = pl.pallas_call(
      _dinv_body,
      out_shape=jax.ShapeDtypeStruct((1250, 128), jnp.float32),
  )(hist2.reshape(2, 1250, 128)).reshape(P)

  xw1 = pl.pallas_call(
      _mm_body,
      grid=(10,),
      in_specs=[pl.BlockSpec((1000, D), lambda i: (i, 0)),
                pl.BlockSpec((D, D), lambda i: (0, 0))],
      out_specs=pl.BlockSpec((1000, D), lambda i: (i, 0)),
      out_shape=jax.ShapeDtypeStruct((N, D), jnp.float32),
  )(x, W1)

  h1, wedge = _l1(dst_all, comp, srcle_all, dinv, xw1, b1)

  z = pl.pallas_call(
      _mm_body,
      grid=(160,),
      in_specs=[pl.BlockSpec((1000, D), lambda i: (i, 0)),
                pl.BlockSpec((D, DCP), lambda i: (0, 0))],
      out_specs=pl.BlockSpec((1000, DCP), lambda i: (i, 0)),
      out_shape=jax.ShapeDtypeStruct((P, DCP), jnp.float32),
  )(h1, W2p)

  out2 = _l2(dst_all, srcle_all, src0, dinv, z, b2p, wedge)

  out = pl.pallas_call(
      _fin_body,
      grid=(16,),
      in_specs=[pl.BlockSpec((640, DCP), lambda i: (i, 0)),
                pl.BlockSpec((640, DCP), lambda i: (i, 0)),
                pl.BlockSpec((640, 1), lambda i: (i, 0)),
                pl.BlockSpec((640, 1), lambda i: (i, 0))],
      out_specs=pl.BlockSpec((640, DCP), lambda i: (i, 0)),
      out_shape=jax.ShapeDtypeStruct((NP, DCP), jnp.float32),
  )(out2[:NP], out2[NP:], cnt2[:NP].reshape(NP, 1), cnt2[NP:].reshape(NP, 1))

  return out[:N, :DC]
